# baseline (speedup 1.0000x reference)
import functools

import jax
import jax.numpy as jnp
from jax import lax
from jax.experimental import pallas as pl
from jax.experimental.pallas import tpu as pltpu
from jax.experimental.pallas import tpu_sc as plsc

NC = 2
NS = 16
NW = NC * NS
C = 128
NBUF = 2


@functools.lru_cache(maxsize=None)
def _build(nchunks, d, vocab):
    mesh = plsc.VectorSubcoreMesh(core_axis_name="c", subcore_axis_name="s")

    @functools.partial(
        pl.kernel,
        mesh=mesh,
        compiler_params=pltpu.CompilerParams(use_tc_tiling_on_sc=False),
        out_type=jax.ShapeDtypeStruct((NW, nchunks, C, d), jnp.float32),
        scratch_types=[
            pltpu.VMEM((nchunks, C), jnp.int32),
            pltpu.VMEM((C, d), jnp.float32),
            pltpu.VMEM((C, d), jnp.float32),
            pltpu.SemaphoreType.DMA,
            pltpu.SemaphoreType.DMA,
        ],
    )
    def emb(table_hbm, idx_hbm, out_hbm, idx_v, buf0, buf1, sem0, sem1):
        wid = lax.axis_index("s") * NC + lax.axis_index("c")
        pltpu.sync_copy(idx_hbm.at[wid], idx_v)
        bufs = (buf0, buf1)
        sems = (sem0, sem1)
        for b in range(NBUF):
            pltpu.async_copy(table_hbm.at[idx_v.at[b]], bufs[b], sems[b])

        def group(gi, carry):
            for b in range(NBUF):
                c = gi * NBUF + b
                pltpu.make_async_copy(
                    table_hbm.at[pl.ds(0, C)], bufs[b], sems[b]
                ).wait()
                pltpu.sync_copy(bufs[b], out_hbm.at[wid].at[c])
                nxt = c + NBUF

                @pl.when(nxt < nchunks)
                def _():
                    pltpu.async_copy(
                        table_hbm.at[idx_v.at[nxt]], bufs[b], sems[b]
                    )
            return carry

        lax.fori_loop(0, nchunks // NBUF, group, 0)

    return emb


def kernel(x, W):
    B, S = x.shape
    V, D = W.shape
    total = B * S
    assert total % (NW * C) == 0
    nchunks = total // (NW * C)
    idx = x.reshape(NW, nchunks, C)
    out = _build(nchunks, D, V)(W, idx)
    return out.reshape(B, S, D)


# speedup vs baseline: 1.0486x; 1.0486x over previous
import functools

import jax
import jax.numpy as jnp
from jax import lax
from jax.experimental import pallas as pl
from jax.experimental.pallas import tpu as pltpu
from jax.experimental.pallas import tpu_sc as plsc

NC = 2
NS = 16
NW = NC * NS
LANE = 128
CH = 126 * LANE


@functools.lru_cache(maxsize=None)
def _build_untile(D, V):
    CPW = D // NW
    vmain = (V // LANE) * LANE
    VP = vmain + LANE
    nch = vmain // CH
    assert nch * CH == vmain and nch % 2 == 0
    mesh = plsc.VectorSubcoreMesh(core_axis_name="c", subcore_axis_name="s")

    @functools.partial(
        pl.kernel,
        mesh=mesh,
        compiler_params=pltpu.CompilerParams(use_tc_tiling_on_sc=True),
        out_type=jax.ShapeDtypeStruct((D * VP,), jnp.float32),
        scratch_types=[
            pltpu.VMEM((CH,), jnp.float32),
            pltpu.VMEM((CH,), jnp.float32),
            pltpu.SemaphoreType.DMA,
            pltpu.SemaphoreType.DMA,
        ],
    )
    def untile(wt_hbm, wtail_hbm, flat_hbm, b0, b1, sem0, sem1):
        wid = lax.axis_index("s") * NC + lax.axis_index("c")
        bufs = (b0, b1)
        sems = (sem0, sem1)

        for cc in range(CPW):
            c = wid * CPW + cc

            def load(i, b):
                pltpu.async_copy(
                    wt_hbm.at[c].at[pl.ds(i * CH, CH)], bufs[b], sems[b]
                )

            def store(i, b):
                pltpu.make_async_copy(
                    flat_hbm.at[pl.ds(0, CH)], bufs[b], sems[b]
                ).wait()
                pltpu.sync_copy(bufs[b], flat_hbm.at[pl.ds(c * VP + i * CH, CH)])

            load(0, 0)
            load(1, 1)

            def body(gi, carry):
                for b in range(2):
                    i = 2 * gi + b
                    store(i, b)

                    @pl.when(i + 2 < nch)
                    def _():
                        load(i + 2, b)
                return carry

            lax.fori_loop(0, nch // 2, body, 0)

            pltpu.sync_copy(wtail_hbm.at[c], bufs[0].at[pl.ds(0, LANE)])
            pltpu.sync_copy(
                bufs[0].at[pl.ds(0, LANE)],
                flat_hbm.at[pl.ds(c * VP + vmain, LANE)],
            )

    return untile


@functools.lru_cache(maxsize=None)
def _build_gather(S, D, B, VP):
    CPW = D // NW
    nrows = CPW * S
    mesh = plsc.VectorSubcoreMesh(core_axis_name="c", subcore_axis_name="s")

    @functools.partial(
        pl.kernel,
        mesh=mesh,
        compiler_params=pltpu.CompilerParams(use_tc_tiling_on_sc=False),
        out_type=jax.ShapeDtypeStruct((S, D, B), jnp.float32),
        scratch_types=[
            pltpu.VMEM((2, B), jnp.int32),
            pltpu.VMEM((B,), jnp.float32),
            pltpu.VMEM((B,), jnp.float32),
            pltpu.SemaphoreType.DMA,
            pltpu.SemaphoreType.DMA,
        ],
    )
    def emb(wt_hbm, idx_hbm, out_hbm, idx_v, g0, g1, sem0, sem1):
        wid = lax.axis_index("s") * NC + lax.axis_index("c")
        c0 = wid * CPW
        bufs = (g0, g1)
        sems = (sem0, sem1)

        def row_sc(r):
            s = r // CPW
            c = c0 + r % CPW
            return s, c

        def start_gather(r, b):
            s, c = row_sc(r)

            @pl.when(r % CPW == 0)
            def _():
                pltpu.sync_copy(idx_hbm.at[s], idx_v.at[s % 2])

            pltpu.async_copy(wt_hbm.at[c].at[idx_v.at[s % 2]], bufs[b], sems[b])

        def drain(r, b):
            pltpu.make_async_copy(
                wt_hbm.at[0].at[pl.ds(0, B)], bufs[b], sems[b]
            ).wait()
            s, c = row_sc(r)
            pltpu.sync_copy(bufs[b], out_hbm.at[s].at[c])

        start_gather(0, 0)

        def body(gi, carry):
            for b in range(2):
                r = 2 * gi + b

                @pl.when(r + 1 < nrows)
                def _():
                    start_gather(r + 1, 1 - b)

                drain(r, b)
            return carry

        lax.fori_loop(0, nrows // 2, body, 0)

    return emb


def kernel(x, W):
    B, S = x.shape
    V, D = W.shape
    VP = (V // LANE) * LANE + LANE
    wt = W.T
    wtail = jnp.pad(W[(V // LANE) * LANE:], ((0, LANE - V % LANE), (0, 0))).T
    flat = _build_untile(D, V)(wt, wtail)
    wt_lin = flat.reshape(D, VP)
    idxt = x.T
    out3 = _build_gather(S, D, B, VP)(wt_lin, idxt)
    return out3.transpose(2, 0, 1)


# speedup vs baseline: 1.0698x; 1.0202x over previous
import functools

import jax
import jax.numpy as jnp
from jax import lax
from jax.experimental import pallas as pl
from jax.experimental.pallas import tpu as pltpu
from jax.experimental.pallas import tpu_sc as plsc

NC = 2
NS = 16
NW = NC * NS
LANE = 128
CH = 126 * LANE


@functools.lru_cache(maxsize=None)
def _build_untile(D, V):
    CPW = D // NW
    vmain = (V // LANE) * LANE
    VP = vmain + LANE
    nch = vmain // CH
    assert nch * CH == vmain and nch % 2 == 0
    mesh = plsc.VectorSubcoreMesh(core_axis_name="c", subcore_axis_name="s")

    @functools.partial(
        pl.kernel,
        mesh=mesh,
        compiler_params=pltpu.CompilerParams(use_tc_tiling_on_sc=True),
        out_type=jax.ShapeDtypeStruct((D * VP,), jnp.float32),
        scratch_types=[
            pltpu.VMEM((CH,), jnp.float32),
            pltpu.VMEM((CH,), jnp.float32),
            pltpu.SemaphoreType.DMA,
            pltpu.SemaphoreType.DMA,
        ],
    )
    def untile(wt_hbm, wtail_hbm, flat_hbm, b0, b1, sem0, sem1):
        wid = lax.axis_index("s") * NC + lax.axis_index("c")
        bufs = (b0, b1)
        sems = (sem0, sem1)

        for cc in range(CPW):
            c = wid * CPW + cc

            def load(i, b):
                pltpu.async_copy(
                    wt_hbm.at[c].at[pl.ds(i * CH, CH)], bufs[b], sems[b]
                )

            def store(i, b):
                pltpu.make_async_copy(
                    flat_hbm.at[pl.ds(0, CH)], bufs[b], sems[b]
                ).wait()
                pltpu.sync_copy(bufs[b], flat_hbm.at[pl.ds(c * VP + i * CH, CH)])

            load(0, 0)
            load(1, 1)

            def body(gi, carry):
                for b in range(2):
                    i = 2 * gi + b
                    store(i, b)

                    @pl.when(i + 2 < nch)
                    def _():
                        load(i + 2, b)
                return carry

            lax.fori_loop(0, nch // 2, body, 0)

            pltpu.sync_copy(wtail_hbm.at[c], bufs[0].at[pl.ds(0, LANE)])
            pltpu.sync_copy(
                bufs[0].at[pl.ds(0, LANE)],
                flat_hbm.at[pl.ds(c * VP + vmain, LANE)],
            )

    return untile


@functools.lru_cache(maxsize=None)
def _build_gather(S, D, B, VP):
    CPW = D // NW
    nrows = CPW * S
    mesh = plsc.VectorSubcoreMesh(core_axis_name="c", subcore_axis_name="s")

    @functools.partial(
        pl.kernel,
        mesh=mesh,
        compiler_params=pltpu.CompilerParams(use_tc_tiling_on_sc=False),
        out_type=jax.ShapeDtypeStruct((S, D, B), jnp.float32),
        scratch_types=[
            pltpu.VMEM((4, B), jnp.int32),
            pltpu.VMEM((4, B), jnp.float32),
            [pltpu.SemaphoreType.DMA] * 4,
        ],
    )
    def emb(wt_hbm, idx_hbm, out_hbm, idx_v, gbuf, sems):
        wid = lax.axis_index("s") * NC + lax.axis_index("c")
        c0 = wid * CPW
        NB = 4

        def row_sc(r):
            s = r // CPW
            c = c0 + r % CPW
            return s, c

        def start_gather(r, b):
            s, c = row_sc(r)

            @pl.when(r % CPW == 0)
            def _():
                pltpu.sync_copy(idx_hbm.at[s], idx_v.at[s % 4])

            pltpu.async_copy(
                wt_hbm.at[c].at[idx_v.at[s % 4]], gbuf.at[b], sems[b]
            )

        def drain(r, b):
            pltpu.make_async_copy(
                wt_hbm.at[0].at[pl.ds(0, B)], gbuf.at[b], sems[b]
            ).wait()
            s, c = row_sc(r)
            pltpu.sync_copy(gbuf.at[b], out_hbm.at[s].at[c])

        for r0 in range(NB - 1):
            start_gather(r0, r0)

        def body(gi, carry):
            for b in range(NB):
                r = NB * gi + b

                @pl.when(r + NB - 1 < nrows)
                def _():
                    start_gather(r + NB - 1, (b + NB - 1) % NB)

                drain(r, b)
            return carry

        lax.fori_loop(0, nrows // NB, body, 0)

    return emb


def kernel(x, W):
    B, S = x.shape
    V, D = W.shape
    VP = (V // LANE) * LANE + LANE
    wt = W.T
    wtail = jnp.pad(W[(V // LANE) * LANE:], ((0, LANE - V % LANE), (0, 0))).T
    flat = _build_untile(D, V)(wt, wtail)
    wt_lin = flat.reshape(D, VP)
    idxt = x.T
    out3 = _build_gather(S, D, B, VP)(wt_lin, idxt)
    return out3.transpose(2, 0, 1)


# speedup vs baseline: 1.1888x; 1.1113x over previous
import functools

import jax
import jax.numpy as jnp
from jax import lax
from jax.experimental import pallas as pl
from jax.experimental.pallas import tpu as pltpu
from jax.experimental.pallas import tpu_sc as plsc

NC = 2
NS = 16
NW = NC * NS
LANE = 128
CH = 126 * LANE


@functools.lru_cache(maxsize=None)
def _build(S, D, B, V):
    CPW = D // NW
    vmain = (V // LANE) * LANE
    VP = vmain + LANE
    nch = vmain // CH
    assert nch * CH == vmain and nch % 2 == 0 and CPW == 2
    mesh = plsc.VectorSubcoreMesh(core_axis_name="c", subcore_axis_name="s")

    @functools.partial(
        pl.kernel,
        mesh=mesh,
        compiler_params=pltpu.CompilerParams(use_tc_tiling_on_sc=True),
        out_type=(
            jax.ShapeDtypeStruct((S, D, B), jnp.float32),
            jax.ShapeDtypeStruct((D * VP,), jnp.float32),
        ),
        scratch_types=[
            [pltpu.VMEM((CH,), jnp.float32)] * 2,
            [pltpu.VMEM((B,), jnp.int32)] * 4,
            [pltpu.VMEM((B,), jnp.float32)] * 4,
            [pltpu.SemaphoreType.DMA] * 2,
            [pltpu.SemaphoreType.DMA] * 4,
        ],
    )
    def emb(wt_hbm, wtail_hbm, idx_hbm, out_hbm, flat_hbm,
            ubuf, idx_v, gbuf, usems, gsems):
        wid = lax.axis_index("s") * NC + lax.axis_index("c")

        def uload(c, i, b):
            pltpu.async_copy(
                wt_hbm.at[c].at[pl.ds(i * CH, CH)], ubuf[b], usems[b]
            )

        def ustore(c, i, b):
            pltpu.make_async_copy(
                flat_hbm.at[pl.ds(0, CH)], ubuf[b], usems[b]
            ).wait()
            off = pl.multiple_of(c * VP + i * CH, LANE)
            pltpu.sync_copy(ubuf[b], flat_hbm.at[pl.ds(off, CH)])

        def utail(c):
            pltpu.sync_copy(wtail_hbm.at[c], ubuf[0].at[pl.ds(0, LANE)])
            off = pl.multiple_of(c * VP + vmain, LANE)
            pltpu.sync_copy(
                ubuf[0].at[pl.ds(0, LANE)],
                flat_hbm.at[pl.ds(off, LANE)],
            )

        def gstart(s, c, b):
            pltpu.sync_copy(idx_hbm.at[s], idx_v[b])
            off = pl.multiple_of(c * VP, LANE)
            pltpu.async_copy(
                flat_hbm.at[pl.ds(off, VP)].at[idx_v[b]],
                gbuf[b], gsems[b],
            )

        def gdrain(s, c, b):
            pltpu.make_async_copy(
                flat_hbm.at[pl.ds(0, B)], gbuf[b], gsems[b]
            ).wait()
            pltpu.sync_copy(gbuf[b], out_hbm.at[s].at[c])

        c0 = wid * CPW
        c1 = c0 + 1

        uload(c0, 0, 0)
        uload(c0, 1, 1)

        def p1(gi, carry):
            for b in range(2):
                i = 2 * gi + b
                ustore(c0, i, b)

                @pl.when(i + 2 < nch)
                def _():
                    uload(c0, i + 2, b)
            return carry

        lax.fori_loop(0, nch // 2, p1, 0)
        utail(c0)

        uload(c1, 0, 0)
        uload(c1, 1, 1)

        def p2(gi, carry):
            for b in range(2):
                m = 2 * gi + b
                ustore(c1, m, b)

                @pl.when(m + 2 < nch)
                def _():
                    uload(c1, m + 2, b)

                @pl.when(m < S)
                def _():
                    gstart(m, c0, b)

                @pl.when(jnp.logical_and(1 <= m, m <= S))
                def _():
                    gdrain(m - 1, c0, 1 - b)
            return carry

        lax.fori_loop(0, nch // 2, p2, 0)
        utail(c1)

        NB = 4
        for r0 in range(NB - 1):
            gstart(r0, c1, r0)

        def p3(gi, carry):
            for b in range(NB):
                r = NB * gi + b

                @pl.when(r + NB - 1 < S)
                def _():
                    gstart(r + NB - 1, c1, (b + NB - 1) % NB)

                @pl.when(r < S)
                def _():
                    gdrain(r, c1, b)
            return carry

        lax.fori_loop(0, (S + NB - 1) // NB, p3, 0)

    return emb


def kernel(x, W):
    B, S = x.shape
    V, D = W.shape
    wt = W.T
    wtail = jnp.pad(W[(V // LANE) * LANE:], ((0, LANE - V % LANE), (0, 0))).T
    idxt = x.T
    out3, _ = _build(S, D, B, V)(wt, wtail, idxt)
    return out3.transpose(2, 0, 1)
